# Initial kernel scaffold; baseline (speedup 1.0000x reference)
import jax
import jax.numpy as jnp
from jax.experimental import pallas as pl


def kernel(x, edge_index, W1, a_src1, a_dst1, b1, W2, a_src2, a_dst2, b2, W3, a_src3, a_dst3, b3):
    raise NotImplementedError("write your pallas kernel here")


# speedup vs baseline: 4.7280x; 4.7280x over previous
import functools

import jax
import jax.numpy as jnp
from jax.experimental import pallas as pl

N = 50000
NB = 1000


def _prep_body(x_ref, w_ref, asrc_ref, adst_ref, h_ref, es_ref, ed_ref, *, heads, ch):
    h = x_ref[...] @ w_ref[...]
    h_ref[...] = h
    h3 = h.reshape(h.shape[0], heads, ch)
    es_ref[...] = (h3 * asrc_ref[...][None]).sum(-1)
    ed_ref[...] = (h3 * adst_ref[...][None]).sum(-1)


def _prep(x, W, a_src, a_dst):
    heads, ch = a_src.shape
    n, din = x.shape
    dout = heads * ch
    grid = (n // NB,)
    return pl.pallas_call(
        functools.partial(_prep_body, heads=heads, ch=ch),
        grid=grid,
        in_specs=[
            pl.BlockSpec((NB, din), lambda i: (i, 0)),
            pl.BlockSpec((din, dout), lambda i: (0, 0)),
            pl.BlockSpec((heads, ch), lambda i: (0, 0)),
            pl.BlockSpec((heads, ch), lambda i: (0, 0)),
        ],
        out_specs=[
            pl.BlockSpec((NB, dout), lambda i: (i, 0)),
            pl.BlockSpec((NB, heads), lambda i: (i, 0)),
            pl.BlockSpec((NB, heads), lambda i: (i, 0)),
        ],
        out_shape=[
            jax.ShapeDtypeStruct((n, dout), jnp.float32),
            jax.ShapeDtypeStruct((n, heads), jnp.float32),
            jax.ShapeDtypeStruct((n, heads), jnp.float32),
        ],
    )(x, W, a_src, a_dst)


def _norm_body(acc_ref, s_ref, b_ref, o_ref, *, act):
    o = acc_ref[...] / (s_ref[...] + 1e-16) + b_ref[...][None]
    if act == "elu":
        o = jnp.where(o > 0, o, jnp.exp(jnp.minimum(o, 0.0)) - 1.0)
    elif act == "logsoftmax":
        m = o.max(axis=1, keepdims=True)
        o = o - (m + jnp.log(jnp.exp(o - m).sum(axis=1, keepdims=True)))
    o_ref[...] = o


def _norm(acc, s, b, act):
    n, d = acc.shape
    grid = (n // NB,)
    return pl.pallas_call(
        functools.partial(_norm_body, act=act),
        grid=grid,
        in_specs=[
            pl.BlockSpec((NB, d), lambda i: (i, 0)),
            pl.BlockSpec((NB, s.shape[1]), lambda i: (i, 0)),
            pl.BlockSpec((d,), lambda i: (0,)),
        ],
        out_specs=pl.BlockSpec((NB, d), lambda i: (i, 0)),
        out_shape=jax.ShapeDtypeStruct((n, d), jnp.float32),
    )(acc, s, b)


def _edge_phase(h, es, ed, src, dst, heads, ch):
    e = es[src] + ed[dst]
    e = jnp.where(e > 0, e, 0.2 * e)
    ex = jnp.exp(e)
    n = h.shape[0]
    msg = h.reshape(-1, heads, ch)[src] * ex[:, :, None]
    acc = jax.ops.segment_sum(msg.reshape(-1, heads * ch), dst, num_segments=n)
    s = jax.ops.segment_sum(ex, dst, num_segments=n)
    return acc, s


def kernel(x, edge_index, W1, a_src1, a_dst1, b1, W2, a_src2, a_dst2, b2,
           W3, a_src3, a_dst3, b3):
    n = x.shape[0]
    loop = jnp.arange(n, dtype=edge_index.dtype)
    src = jnp.concatenate([edge_index[0], loop])
    dst = jnp.concatenate([edge_index[1], loop])

    h, es, ed = _prep(x, W1, a_src1, a_dst1)
    acc, s = _edge_phase(h, es, ed, src, dst, 4, 16)
    h = _norm(acc, jnp.repeat(s, 16, axis=1), b1, "elu")

    h, es, ed = _prep(h, W2, a_src2, a_dst2)
    acc, s = _edge_phase(h, es, ed, src, dst, 4, 16)
    h = _norm(acc, jnp.repeat(s, 16, axis=1), b2, "elu")

    h, es, ed = _prep(h, W3, a_src3, a_dst3)
    acc, s = _edge_phase(h, es, ed, src, dst, 1, 2)
    return _norm(acc, s, b3, "logsoftmax")


# speedup vs baseline: 57.6361x; 12.1904x over previous
import functools

import jax
import jax.numpy as jnp
from jax import lax
from jax.experimental import pallas as pl
from jax.experimental.pallas import tpu as pltpu
from jax.experimental.pallas import tpu_sc as plsc

N = 50000
E_REAL = 850000
K = 128
EP = 851968
T = EP // K
NB = 1000
NTA = 3128


def _iota16():
    return lax.iota(jnp.int32, 16)


def _splat(v):
    return jnp.full((16,), v, jnp.int32)


def _zero_accum(si, row_v, accum):
    base = pl.multiple_of(si * NTA, 8)

    def zinit(i, carry):
        pltpu.sync_copy(row_v, accum.at[pl.ds(base + i * K, K)])
        return carry
    lax.fori_loop(0, NTA // K, zinit, 0, unroll=False)

    @pl.when(si < 15)
    def _():
        pltpu.sync_copy(row_v.at[pl.ds(0, 56)],
                        accum.at[pl.ds(base + 3072, 56)])

    @pl.when(si == 15)
    def _():
        pltpu.sync_copy(row_v.at[pl.ds(0, 8)],
                        accum.at[pl.ds(base + 3072, 8)])


def _writeback(ci, si, accum, out0, out1):
    base = pl.multiple_of(si * NTA, 8)
    last = N - 15 * NTA
    for cval, ref in ((0, out0), (1, out1)):
        @pl.when((ci == cval) & (si < 15))
        def _(ref=ref):
            pltpu.sync_copy(accum.at[pl.ds(base, NTA)],
                            ref.at[pl.ds(base, NTA)])

        @pl.when((ci == cval) & (si == 15))
        def _(ref=ref):
            pltpu.sync_copy(accum.at[pl.ds(base, last)],
                            ref.at[pl.ds(base, last)])


def _sc12_body(src_hbm, dst_hbm, attn_hbm, hlo_hbm, hhi_hbm,
               out0_hbm, out1_hbm,
               src_v, dst_v, as_v, ad_v, h_v, row_v, accum,
               sem0, sem1, sem2):
    ci = lax.axis_index("c")
    si = lax.axis_index("s")
    zero = jnp.zeros((16,), jnp.float32)
    iota = _iota16()

    def zrow(i, carry):
        for c in range(2):
            row_v[i, pl.ds(c * 16, 16)] = zero
        return carry
    lax.fori_loop(0, K, zrow, 0, unroll=False)

    _zero_accum(si, row_v, accum)
    plsc.subcore_barrier()

    def chunk(k, carry):
        t = si + k * 16
        off = pl.multiple_of(t * K, 8)
        pltpu.sync_copy(src_hbm.at[pl.ds(off, K)], src_v)
        pltpu.sync_copy(dst_hbm.at[pl.ds(off, K)], dst_v)
        cp0 = pltpu.async_copy(attn_hbm.at[src_v], as_v, sem0)
        cp1 = pltpu.async_copy(attn_hbm.at[dst_v], ad_v, sem1)

        @pl.when(ci == 0)
        def _():
            pltpu.async_copy(hlo_hbm.at[src_v], h_v, sem2).wait()

        @pl.when(ci != 0)
        def _():
            pltpu.async_copy(hhi_hbm.at[src_v], h_v, sem2).wait()

        cp0.wait()
        cp1.wait()

        shift4 = jnp.minimum(iota + 4, 15)
        i0 = _splat(2 * ci)
        i1 = _splat(2 * ci + 1)
        for jj in range(K):
            s_row = as_v[jj, pl.ds(0, 16)]
            d_row = ad_v[jj, pl.ds(0, 16)]
            e = s_row + d_row.at[shift4].get(mode="promise_in_bounds")
            e = jnp.maximum(e, 0.2 * e)
            vf = (off + jj < E_REAL).astype(jnp.float32)
            ex4 = jnp.exp(e) * jnp.full((16,), vf, jnp.float32)
            exb0 = ex4.at[i0].get(mode="promise_in_bounds")
            exb1 = ex4.at[i1].get(mode="promise_in_bounds")
            row_v[jj, pl.ds(0, 16)] = h_v[jj, pl.ds(0, 16)] * exb0
            row_v[jj, pl.ds(16, 16)] = h_v[jj, pl.ds(16, 16)] * exb1
        pltpu.sync_copy(row_v, accum.at[dst_v], add=True)
        return carry

    lax.fori_loop(0, T // 16, chunk, 0, unroll=False)
    plsc.subcore_barrier()
    _writeback(ci, si, accum, out0_hbm, out1_hbm)


def _sc12(src, dst, attn, hlo, hhi):
    mesh = plsc.VectorSubcoreMesh(core_axis_name="c", subcore_axis_name="s")
    f = pl.kernel(
        _sc12_body,
        out_type=[jax.ShapeDtypeStruct((N, 32), jnp.float32),
                  jax.ShapeDtypeStruct((N, 32), jnp.float32)],
        mesh=mesh,
        compiler_params=pltpu.CompilerParams(use_tc_tiling_on_sc=False),
        scratch_types=[
            pltpu.VMEM((K,), jnp.int32),
            pltpu.VMEM((K,), jnp.int32),
            pltpu.VMEM((K, 16), jnp.float32),
            pltpu.VMEM((K, 16), jnp.float32),
            pltpu.VMEM((K, 32), jnp.float32),
            pltpu.VMEM((K, 32), jnp.float32),
            pltpu.VMEM_SHARED((N, 32), jnp.float32),
            pltpu.SemaphoreType.DMA,
            pltpu.SemaphoreType.DMA,
            pltpu.SemaphoreType.DMA,
        ],
    )
    return f(src, dst, attn, hlo, hhi)


def _scs_body(src_hbm, dst_hbm, attn_hbm,
              out0_hbm, out1_hbm,
              src_v, dst_v, as_v, ad_v, row_v, accum, sem0, sem1):
    ci = lax.axis_index("c")
    si = lax.axis_index("s")
    zero = jnp.zeros((16,), jnp.float32)
    iota = _iota16()

    def zrow(i, carry):
        row_v[i, pl.ds(0, 16)] = zero
        return carry
    lax.fori_loop(0, K, zrow, 0, unroll=False)

    _zero_accum(si, row_v, accum)
    plsc.subcore_barrier()

    w = si * 2 + ci

    def chunk(k, carry):
        t = w + k * 32
        off = pl.multiple_of(t * K, 8)
        pltpu.sync_copy(src_hbm.at[pl.ds(off, K)], src_v)
        pltpu.sync_copy(dst_hbm.at[pl.ds(off, K)], dst_v)
        cp0 = pltpu.async_copy(attn_hbm.at[src_v], as_v, sem0)
        cp1 = pltpu.async_copy(attn_hbm.at[dst_v], ad_v, sem1)
        cp0.wait()
        cp1.wait()

        shift4 = jnp.minimum(iota + 4, 15)
        m4 = (1 - jnp.minimum(iota >> 2, 1)).astype(jnp.float32)
        for jj in range(K):
            s_row = as_v[jj, pl.ds(0, 16)]
            d_row = ad_v[jj, pl.ds(0, 16)]
            e = s_row + d_row.at[shift4].get(mode="promise_in_bounds")
            e = jnp.maximum(e, 0.2 * e)
            vf = (off + jj < E_REAL).astype(jnp.float32)
            row_v[jj, pl.ds(0, 16)] = jnp.exp(e) * (
                m4 * jnp.full((16,), vf, jnp.float32))
        pltpu.sync_copy(row_v, accum.at[dst_v], add=True)
        return carry

    lax.fori_loop(0, T // 32, chunk, 0, unroll=False)
    plsc.subcore_barrier()
    _writeback(ci, si, accum, out0_hbm, out1_hbm)


def _scs(src, dst, attn):
    mesh = plsc.VectorSubcoreMesh(core_axis_name="c", subcore_axis_name="s")
    f = pl.kernel(
        _scs_body,
        out_type=[jax.ShapeDtypeStruct((N, 16), jnp.float32),
                  jax.ShapeDtypeStruct((N, 16), jnp.float32)],
        mesh=mesh,
        compiler_params=pltpu.CompilerParams(use_tc_tiling_on_sc=False),
        scratch_types=[
            pltpu.VMEM((K,), jnp.int32),
            pltpu.VMEM((K,), jnp.int32),
            pltpu.VMEM((K, 16), jnp.float32),
            pltpu.VMEM((K, 16), jnp.float32),
            pltpu.VMEM((K, 16), jnp.float32),
            pltpu.VMEM_SHARED((N, 16), jnp.float32),
            pltpu.SemaphoreType.DMA,
            pltpu.SemaphoreType.DMA,
        ],
    )
    return f(src, dst, attn)


def _sc3_body(src_hbm, dst_hbm, attn_hbm, h_hbm,
              out0_hbm, out1_hbm,
              src_v, dst_v, as_v, ad_v, h_v, row_v, accum,
              sem0, sem1, sem2):
    ci = lax.axis_index("c")
    si = lax.axis_index("s")
    zero = jnp.zeros((16,), jnp.float32)
    iota = _iota16()

    def zrow(i, carry):
        row_v[i, pl.ds(0, 16)] = zero
        return carry
    lax.fori_loop(0, K, zrow, 0, unroll=False)

    _zero_accum(si, row_v, accum)
    plsc.subcore_barrier()

    w = si * 2 + ci

    def chunk(k, carry):
        t = w + k * 32
        off = pl.multiple_of(t * K, 8)
        pltpu.sync_copy(src_hbm.at[pl.ds(off, K)], src_v)
        pltpu.sync_copy(dst_hbm.at[pl.ds(off, K)], dst_v)
        cp0 = pltpu.async_copy(attn_hbm.at[src_v], as_v, sem0)
        cp1 = pltpu.async_copy(attn_hbm.at[dst_v], ad_v, sem1)
        cp2 = pltpu.async_copy(h_hbm.at[src_v], h_v, sem2)
        cp0.wait()
        cp1.wait()
        cp2.wait()

        shift1 = jnp.minimum(iota + 1, 15)
        izero = _splat(0)
        for jj in range(K):
            s_row = as_v[jj, pl.ds(0, 16)]
            d_row = ad_v[jj, pl.ds(0, 16)]
            e = s_row + d_row.at[shift1].get(mode="promise_in_bounds")
            e = jnp.maximum(e, 0.2 * e)
            vf = (off + jj < E_REAL).astype(jnp.float32)
            ex = jnp.exp(e) * jnp.full((16,), vf, jnp.float32)
            exb = ex.at[izero].get(mode="promise_in_bounds")
            row_v[jj, pl.ds(0, 16)] = h_v[jj, pl.ds(0, 16)] * exb
        pltpu.sync_copy(row_v, accum.at[dst_v], add=True)
        return carry

    lax.fori_loop(0, T // 32, chunk, 0, unroll=False)
    plsc.subcore_barrier()
    _writeback(ci, si, accum, out0_hbm, out1_hbm)


def _sc3(src, dst, attn, hp):
    mesh = plsc.VectorSubcoreMesh(core_axis_name="c", subcore_axis_name="s")
    f = pl.kernel(
        _sc3_body,
        out_type=[jax.ShapeDtypeStruct((N, 16), jnp.float32),
                  jax.ShapeDtypeStruct((N, 16), jnp.float32)],
        mesh=mesh,
        compiler_params=pltpu.CompilerParams(use_tc_tiling_on_sc=False),
        scratch_types=[
            pltpu.VMEM((K,), jnp.int32),
            pltpu.VMEM((K,), jnp.int32),
            pltpu.VMEM((K, 16), jnp.float32),
            pltpu.VMEM((K, 16), jnp.float32),
            pltpu.VMEM((K, 16), jnp.float32),
            pltpu.VMEM((K, 16), jnp.float32),
            pltpu.VMEM_SHARED((N, 16), jnp.float32),
            pltpu.SemaphoreType.DMA,
            pltpu.SemaphoreType.DMA,
            pltpu.SemaphoreType.DMA,
        ],
    )
    return f(src, dst, attn, hp)


def _prep1_body(x_ref, w_ref, asrc_ref, adst_ref, hlo_ref, hhi_ref, attn_ref):
    h = x_ref[...] @ w_ref[...]
    hlo_ref[...] = h[:, :32]
    hhi_ref[...] = h[:, 32:]
    h3 = h.reshape(h.shape[0], 4, 16)
    es = (h3 * asrc_ref[...][None]).sum(-1)
    ed = (h3 * adst_ref[...][None]).sum(-1)
    attn_ref[...] = jnp.concatenate(
        [es, ed, jnp.zeros((h.shape[0], 8), jnp.float32)], axis=1)


def _prep1(x, W, a_src, a_dst):
    grid = (N // NB,)
    return pl.pallas_call(
        _prep1_body,
        grid=grid,
        in_specs=[
            pl.BlockSpec((NB, x.shape[1]), lambda i: (i, 0)),
            pl.BlockSpec((x.shape[1], 64), lambda i: (0, 0)),
            pl.BlockSpec((4, 16), lambda i: (0, 0)),
            pl.BlockSpec((4, 16), lambda i: (0, 0)),
        ],
        out_specs=[
            pl.BlockSpec((NB, 32), lambda i: (i, 0)),
            pl.BlockSpec((NB, 32), lambda i: (i, 0)),
            pl.BlockSpec((NB, 16), lambda i: (i, 0)),
        ],
        out_shape=[
            jax.ShapeDtypeStruct((N, 32), jnp.float32),
            jax.ShapeDtypeStruct((N, 32), jnp.float32),
            jax.ShapeDtypeStruct((N, 16), jnp.float32),
        ],
    )(x, W, a_src, a_dst)


def _elu(o):
    return jnp.where(o > 0, o, jnp.exp(jnp.minimum(o, 0.0)) - 1.0)


def _combine12(o0, o1, sacc, b_ref):
    eps = 1e-16
    parts = [
        o0[:, 0:16] / (sacc[:, 0:1] + eps),
        o0[:, 16:32] / (sacc[:, 1:2] + eps),
        o1[:, 0:16] / (sacc[:, 2:3] + eps),
        o1[:, 16:32] / (sacc[:, 3:4] + eps),
    ]
    h = jnp.concatenate(parts, axis=1) + b_ref[...][None]
    return _elu(h)


def _prep2_body(o0_ref, o1_ref, s0_ref, s1_ref, bprev_ref, w_ref,
                asrc_ref, adst_ref, hlo_ref, hhi_ref, attn_ref):
    hin = _combine12(o0_ref[...], o1_ref[...], s0_ref[...] + s1_ref[...],
                     bprev_ref)
    h = hin @ w_ref[...]
    hlo_ref[...] = h[:, :32]
    hhi_ref[...] = h[:, 32:]
    h3 = h.reshape(h.shape[0], 4, 16)
    es = (h3 * asrc_ref[...][None]).sum(-1)
    ed = (h3 * adst_ref[...][None]).sum(-1)
    attn_ref[...] = jnp.concatenate(
        [es, ed, jnp.zeros((h.shape[0], 8), jnp.float32)], axis=1)


def _prep2(o0, o1, s0, s1, b_prev, W, a_src, a_dst):
    grid = (N // NB,)
    return pl.pallas_call(
        _prep2_body,
        grid=grid,
        in_specs=[
            pl.BlockSpec((NB, 32), lambda i: (i, 0)),
            pl.BlockSpec((NB, 32), lambda i: (i, 0)),
            pl.BlockSpec((NB, 16), lambda i: (i, 0)),
            pl.BlockSpec((NB, 16), lambda i: (i, 0)),
            pl.BlockSpec((64,), lambda i: (0,)),
            pl.BlockSpec((64, 64), lambda i: (0, 0)),
            pl.BlockSpec((4, 16), lambda i: (0, 0)),
            pl.BlockSpec((4, 16), lambda i: (0, 0)),
        ],
        out_specs=[
            pl.BlockSpec((NB, 32), lambda i: (i, 0)),
            pl.BlockSpec((NB, 32), lambda i: (i, 0)),
            pl.BlockSpec((NB, 16), lambda i: (i, 0)),
        ],
        out_shape=[
            jax.ShapeDtypeStruct((N, 32), jnp.float32),
            jax.ShapeDtypeStruct((N, 32), jnp.float32),
            jax.ShapeDtypeStruct((N, 16), jnp.float32),
        ],
    )(o0, o1, s0, s1, b_prev, W, a_src, a_dst)


def _prep3_body(o0_ref, o1_ref, s0_ref, s1_ref, bprev_ref, w_ref,
                asrc_ref, adst_ref, hp_ref, attn_ref):
    hin = _combine12(o0_ref[...], o1_ref[...], s0_ref[...] + s1_ref[...],
                     bprev_ref)
    h = hin @ w_ref[...]
    nb = h.shape[0]
    es = h @ asrc_ref[...].reshape(2, 1)
    ed = h @ adst_ref[...].reshape(2, 1)
    z13 = jnp.zeros((nb, 13), jnp.float32)
    one = jnp.ones((nb, 1), jnp.float32)
    hp_ref[...] = jnp.concatenate([h, one, z13], axis=1)
    attn_ref[...] = jnp.concatenate(
        [es, ed, jnp.zeros((nb, 14), jnp.float32)], axis=1)


def _prep3(o0, o1, s0, s1, b_prev, W, a_src, a_dst):
    grid = (N // NB,)
    return pl.pallas_call(
        _prep3_body,
        grid=grid,
        in_specs=[
            pl.BlockSpec((NB, 32), lambda i: (i, 0)),
            pl.BlockSpec((NB, 32), lambda i: (i, 0)),
            pl.BlockSpec((NB, 16), lambda i: (i, 0)),
            pl.BlockSpec((NB, 16), lambda i: (i, 0)),
            pl.BlockSpec((64,), lambda i: (0,)),
            pl.BlockSpec((64, 2), lambda i: (0, 0)),
            pl.BlockSpec((1, 2), lambda i: (0, 0)),
            pl.BlockSpec((1, 2), lambda i: (0, 0)),
        ],
        out_specs=[
            pl.BlockSpec((NB, 16), lambda i: (i, 0)),
            pl.BlockSpec((NB, 16), lambda i: (i, 0)),
        ],
        out_shape=[
            jax.ShapeDtypeStruct((N, 16), jnp.float32),
            jax.ShapeDtypeStruct((N, 16), jnp.float32),
        ],
    )(o0, o1, s0, s1, b_prev, W, a_src, a_dst)


def _final_body(o0_ref, o1_ref, b_ref, out_ref):
    acc = o0_ref[...][:, :3] + o1_ref[...][:, :3]
    o = acc[:, :2] / (acc[:, 2:3] + 1e-16) + b_ref[...][None]
    m = o.max(axis=1, keepdims=True)
    out_ref[...] = o - (m + jnp.log(jnp.exp(o - m).sum(axis=1, keepdims=True)))


def _final(o0, o1, b3):
    grid = (N // NB,)
    return pl.pallas_call(
        _final_body,
        grid=grid,
        in_specs=[
            pl.BlockSpec((NB, 16), lambda i: (i, 0)),
            pl.BlockSpec((NB, 16), lambda i: (i, 0)),
            pl.BlockSpec((2,), lambda i: (0,)),
        ],
        out_specs=pl.BlockSpec((NB, 2), lambda i: (i, 0)),
        out_shape=jax.ShapeDtypeStruct((N, 2), jnp.float32),
    )(o0, o1, b3)


def kernel(x, edge_index, W1, a_src1, a_dst1, b1, W2, a_src2, a_dst2, b2,
           W3, a_src3, a_dst3, b3):
    loop = jnp.arange(N, dtype=edge_index.dtype)
    padz = jnp.zeros((EP - E_REAL,), edge_index.dtype)
    src = jnp.concatenate([edge_index[0], loop, padz])
    dst = jnp.concatenate([edge_index[1], loop, padz])

    hlo, hhi, attn = _prep1(x, W1, a_src1, a_dst1)
    o0, o1 = _sc12(src, dst, attn, hlo, hhi)
    s0, s1 = _scs(src, dst, attn)
    hlo, hhi, attn = _prep2(o0, o1, s0, s1, b1, W2, a_src2, a_dst2)
    o0, o1 = _sc12(src, dst, attn, hlo, hhi)
    s0, s1 = _scs(src, dst, attn)
    hp, attn = _prep3(o0, o1, s0, s1, b2, W3, a_src3, a_dst3)
    o0, o1 = _sc3(src, dst, attn, hp)
    return _final(o0, o1, b3)
